# baseline (speedup 1.0000x reference)
import jax
import jax.numpy as jnp
from jax.experimental import pallas as pl
from jax.experimental.pallas import tpu as pltpu


def _body(idx_ref, h_ref, cv_ref, o_ref):
    o_ref[...] = h_ref[...] + cv_ref[0]


def kernel(hidden_states, affective_state_indices, control_vectors):
    B, S, E = hidden_states.shape
    n = control_vectors.shape[0]
    idx = affective_state_indices.astype(jnp.int32)
    cv3 = control_vectors.reshape(n, 1, E)
    BS = 512
    grid = (B, S // BS)

    def h_map(b, s, idx_ref):
        return (b, s, 0)

    def cv_map(b, s, idx_ref):
        return (jnp.clip(idx_ref[b], 0, n - 1), 0, 0)

    return pl.pallas_call(
        _body,
        grid_spec=pltpu.PrefetchScalarGridSpec(
            num_scalar_prefetch=1,
            grid=grid,
            in_specs=[
                pl.BlockSpec((1, BS, E), h_map),
                pl.BlockSpec((1, 1, E), cv_map),
            ],
            out_specs=pl.BlockSpec((1, BS, E), h_map),
        ),
        out_shape=jax.ShapeDtypeStruct((B, S, E), hidden_states.dtype),
    )(idx, hidden_states, cv3)


# speedup vs baseline: 1.1010x; 1.1010x over previous
import jax
import jax.numpy as jnp
from jax.experimental import pallas as pl
from jax.experimental.pallas import tpu as pltpu


def _body(idx_ref, h_ref, cv_ref, o_ref):
    o_ref[...] = h_ref[...] + cv_ref[0]


def kernel(hidden_states, affective_state_indices, control_vectors):
    B, S, E = hidden_states.shape
    n = control_vectors.shape[0]
    idx = affective_state_indices.astype(jnp.int32)
    cv3 = control_vectors.reshape(n, 1, E)
    BS = 2048
    grid = (B, S // BS)

    def h_map(b, s, idx_ref):
        return (b, s, 0)

    def cv_map(b, s, idx_ref):
        return (jnp.clip(idx_ref[b], 0, n - 1), 0, 0)

    return pl.pallas_call(
        _body,
        grid_spec=pltpu.PrefetchScalarGridSpec(
            num_scalar_prefetch=1,
            grid=grid,
            in_specs=[
                pl.BlockSpec((1, BS, E), h_map),
                pl.BlockSpec((1, 1, E), cv_map),
            ],
            out_specs=pl.BlockSpec((1, BS, E), h_map),
        ),
        out_shape=jax.ShapeDtypeStruct((B, S, E), hidden_states.dtype),
    )(idx, hidden_states, cv3)
